# baseline (speedup 1.0000x reference)
import functools

import jax
import jax.numpy as jnp
from jax import lax
from jax.experimental import pallas as pl
from jax.experimental.pallas import tpu as pltpu
from jax.experimental.pallas import tpu_sc as plsc

LENGTH = 1000000
BATCH = 16384
L = 16
NC = 2
NS = 16
NW = NC * NS
B_PER_W = BATCH // NW
IDX_CHUNK = 128
N_CHUNKS = B_PER_W // IDX_CHUNK
GROUPS = B_PER_W // L


def _rsqrt(x):
    i = lax.bitcast_convert_type(x, jnp.int32)
    i = jnp.int32(0x5F3759DF) - lax.shift_right_logical(i, 1)
    y = lax.bitcast_convert_type(i, jnp.float32)
    for _ in range(3):
        y = y * (1.5 - 0.5 * x * y * y)
    return y


def _pose_body(table_hbm, idx_hbm, out_hbm, idx_v, idx9_v, cols_v, out_v, sem):
    wid = lax.axis_index("s") * NC + lax.axis_index("c")
    base = wid * B_PER_W

    pltpu.sync_copy(idx_hbm.at[pl.ds(wid * N_CHUNKS, N_CHUNKS)], idx_v)

    for j in range(N_CHUNKS):
        for k in range(IDX_CHUNK // L):
            s = pl.ds(k * L, L)
            v = idx_v[j, s]
            for c in range(9):
                idx9_v[c * N_CHUNKS + j, s] = v + c * LENGTH

    copies = []
    for c in range(9):
        for j in range(N_CHUNKS):
            copies.append(
                pltpu.async_copy(
                    table_hbm.at[idx9_v.at[c * N_CHUNKS + j]],
                    cols_v.at[pl.ds(c * B_PER_W + j * IDX_CHUNK, IDX_CHUNK)],
                    sem,
                )
            )
    for cp in copies:
        cp.wait()

    def group(i, _):
        def col(c):
            return cols_v[pl.ds(c * B_PER_W + i * L, L)]

        t0, t1, t2 = col(0), col(1), col(2)
        a10, a11, a12 = col(3), col(4), col(5)
        a20, a21, a22 = col(6), col(7), col(8)

        n1 = jnp.maximum(a10 * a10 + a11 * a11 + a12 * a12, 1e-24)
        s1 = _rsqrt(n1)
        b10, b11, b12 = a10 * s1, a11 * s1, a12 * s1

        d = b10 * a20 + b11 * a21 + b12 * a22
        u0, u1, u2 = a20 - d * b10, a21 - d * b11, a22 - d * b12
        n2 = jnp.maximum(u0 * u0 + u1 * u1 + u2 * u2, 1e-24)
        s2 = _rsqrt(n2)
        b20, b21, b22 = u0 * s2, u1 * s2, u2 * s2

        b30 = b11 * b22 - b12 * b21
        b31 = b12 * b20 - b10 * b22
        b32 = b10 * b21 - b11 * b20

        outs = (b10, b11, b12, t0, b20, b21, b22, t1, b30, b31, b32, t2)
        for c, v in enumerate(outs):
            out_v[pl.ds(c * B_PER_W + i * L, L)] = v
        return _

    lax.fori_loop(0, GROUPS, group, None)

    pltpu.sync_copy(out_v, out_hbm.at[pl.ds(wid * 12 * B_PER_W, 12 * B_PER_W)])


@jax.jit
def _pose_kernel(table_flat, idx2d):
    mesh = plsc.VectorSubcoreMesh(core_axis_name="c", subcore_axis_name="s")
    return pl.kernel(
        _pose_body,
        out_type=jax.ShapeDtypeStruct((BATCH * 12,), jnp.float32),
        mesh=mesh,
        scratch_types=[
            pltpu.VMEM((N_CHUNKS, IDX_CHUNK), jnp.int32),
            pltpu.VMEM((9 * N_CHUNKS, IDX_CHUNK), jnp.int32),
            pltpu.VMEM((9 * B_PER_W,), jnp.float32),
            pltpu.VMEM((12 * B_PER_W,), jnp.float32),
            pltpu.SemaphoreType.DMA,
        ],
    )(table_flat, idx2d)


def kernel(poses_embed, pose_indices):
    idx2d = pose_indices.astype(jnp.int32).reshape(BATCH // IDX_CHUNK, IDX_CHUNK)
    out = _pose_kernel(poses_embed.T.reshape(9 * LENGTH), idx2d)
    out = out.reshape(NW, 12, B_PER_W).transpose(0, 2, 1)
    return out.reshape(BATCH, 3, 4)


# speedup vs baseline: 11.9093x; 11.9093x over previous
import functools

import jax
import jax.numpy as jnp
from jax import lax
from jax.experimental import pallas as pl
from jax.experimental.pallas import tpu as pltpu
from jax.experimental.pallas import tpu_sc as plsc

LENGTH = 1000000
BATCH = 16384
L = 16
NC = 2
NS = 16
NW = NC * NS
B_PER_W = BATCH // NW
IDX_CHUNK = 128
N_CHUNKS = B_PER_W // IDX_CHUNK
GROUPS = B_PER_W // L

DETILE_BLK = 65536


def _rsqrt(x):
    i = lax.bitcast_convert_type(x, jnp.int32)
    i = jnp.int32(0x5F3759DF) - lax.shift_right_logical(i, 1)
    y = lax.bitcast_convert_type(i, jnp.float32)
    for _ in range(3):
        y = y * (1.5 - 0.5 * x * y * y)
    return y


def _detile_body(in_ref, *out_refs):
    for c in range(9):
        out_refs[c][...] = in_ref[c, :]


def _split_components(table_t):
    grid = (LENGTH + DETILE_BLK - 1) // DETILE_BLK
    return pl.pallas_call(
        _detile_body,
        grid=(grid,),
        in_specs=[pl.BlockSpec((9, DETILE_BLK), lambda j: (0, j))],
        out_specs=[pl.BlockSpec((DETILE_BLK,), lambda j: (j,))] * 9,
        out_shape=[jax.ShapeDtypeStruct((LENGTH,), jnp.float32)] * 9,
    )(table_t)


def _pose_body(*refs):
    tables = refs[:9]
    idx_hbm, out_hbm, idx_v, cols_v, out_v, sem = refs[9:]
    wid = lax.axis_index("s") * NC + lax.axis_index("c")

    pltpu.sync_copy(idx_hbm.at[pl.ds(wid * N_CHUNKS, N_CHUNKS)], idx_v)

    copies = []
    for c in range(9):
        for j in range(N_CHUNKS):
            copies.append(
                pltpu.async_copy(
                    tables[c].at[idx_v.at[j]],
                    cols_v.at[pl.ds(c * B_PER_W + j * IDX_CHUNK, IDX_CHUNK)],
                    sem,
                )
            )
    for cp in copies:
        cp.wait()

    def group(i, _):
        def col(c):
            return cols_v[pl.ds(c * B_PER_W + i * L, L)]

        t0, t1, t2 = col(0), col(1), col(2)
        a10, a11, a12 = col(3), col(4), col(5)
        a20, a21, a22 = col(6), col(7), col(8)

        n1 = jnp.maximum(a10 * a10 + a11 * a11 + a12 * a12, 1e-24)
        s1 = _rsqrt(n1)
        b10, b11, b12 = a10 * s1, a11 * s1, a12 * s1

        d = b10 * a20 + b11 * a21 + b12 * a22
        u0, u1, u2 = a20 - d * b10, a21 - d * b11, a22 - d * b12
        n2 = jnp.maximum(u0 * u0 + u1 * u1 + u2 * u2, 1e-24)
        s2 = _rsqrt(n2)
        b20, b21, b22 = u0 * s2, u1 * s2, u2 * s2

        b30 = b11 * b22 - b12 * b21
        b31 = b12 * b20 - b10 * b22
        b32 = b10 * b21 - b11 * b20

        outs = (b10, b11, b12, t0, b20, b21, b22, t1, b30, b31, b32, t2)
        for c, v in enumerate(outs):
            out_v[pl.ds(c * B_PER_W + i * L, L)] = v
        return _

    lax.fori_loop(0, GROUPS, group, None)

    pltpu.sync_copy(out_v, out_hbm.at[pl.ds(wid * 12 * B_PER_W, 12 * B_PER_W)])


@jax.jit
def _pose_kernel(table, idx2d):
    comps = _split_components(table.T)
    mesh = plsc.VectorSubcoreMesh(core_axis_name="c", subcore_axis_name="s")
    return pl.kernel(
        _pose_body,
        out_type=jax.ShapeDtypeStruct((BATCH * 12,), jnp.float32),
        mesh=mesh,
        scratch_types=[
            pltpu.VMEM((N_CHUNKS, IDX_CHUNK), jnp.int32),
            pltpu.VMEM((9 * B_PER_W,), jnp.float32),
            pltpu.VMEM((12 * B_PER_W,), jnp.float32),
            pltpu.SemaphoreType.DMA,
        ],
    )(*comps, idx2d)


def kernel(poses_embed, pose_indices):
    idx2d = pose_indices.astype(jnp.int32).reshape(BATCH // IDX_CHUNK, IDX_CHUNK)
    out = _pose_kernel(poses_embed, idx2d)
    out = out.reshape(NW, 12, B_PER_W).transpose(0, 2, 1)
    return out.reshape(BATCH, 3, 4)


# speedup vs baseline: 12.2125x; 1.0255x over previous
import functools

import jax
import jax.numpy as jnp
from jax import lax
from jax.experimental import pallas as pl
from jax.experimental.pallas import tpu as pltpu
from jax.experimental.pallas import tpu_sc as plsc

LENGTH = 1000000
BATCH = 16384
L = 16
NC = 2
NS = 16
NW = NC * NS
B_PER_W = BATCH // NW
IDX_CHUNK = 128
N_CHUNKS = B_PER_W // IDX_CHUNK
GROUPS = B_PER_W // L

DETILE_BLK = 131072


def _rsqrt(x):
    i = lax.bitcast_convert_type(x, jnp.int32)
    i = jnp.int32(0x5F3759DF) - lax.shift_right_logical(i, 1)
    y = lax.bitcast_convert_type(i, jnp.float32)
    for _ in range(3):
        y = y * (1.5 - 0.5 * x * y * y)
    return y


def _detile_body(in_ref, *out_refs):
    for c in range(9):
        out_refs[c][...] = in_ref[c, :]


def _split_components(table_t):
    grid = (LENGTH + DETILE_BLK - 1) // DETILE_BLK
    return pl.pallas_call(
        _detile_body,
        grid=(grid,),
        in_specs=[pl.BlockSpec((9, DETILE_BLK), lambda j: (0, j))],
        out_specs=[pl.BlockSpec((DETILE_BLK,), lambda j: (j,))] * 9,
        out_shape=[jax.ShapeDtypeStruct((LENGTH,), jnp.float32)] * 9,
    )(table_t)


def _pose_body(*refs):
    tables = refs[:9]
    idx_hbm, out_hbm, idx_v, cols_v, out_v, sem = refs[9:]
    wid = lax.axis_index("s") * NC + lax.axis_index("c")

    pltpu.sync_copy(idx_hbm.at[pl.ds(wid * N_CHUNKS, N_CHUNKS)], idx_v)

    copies = []
    for c in range(9):
        for j in range(N_CHUNKS):
            copies.append(
                pltpu.async_copy(
                    tables[c].at[idx_v.at[j]],
                    cols_v.at[pl.ds(c * B_PER_W + j * IDX_CHUNK, IDX_CHUNK)],
                    sem,
                )
            )
    for cp in copies:
        cp.wait()

    def group(i, _):
        def col(c):
            return cols_v[pl.ds(c * B_PER_W + i * L, L)]

        t0, t1, t2 = col(0), col(1), col(2)
        a10, a11, a12 = col(3), col(4), col(5)
        a20, a21, a22 = col(6), col(7), col(8)

        n1 = jnp.maximum(a10 * a10 + a11 * a11 + a12 * a12, 1e-24)
        s1 = _rsqrt(n1)
        b10, b11, b12 = a10 * s1, a11 * s1, a12 * s1

        d = b10 * a20 + b11 * a21 + b12 * a22
        u0, u1, u2 = a20 - d * b10, a21 - d * b11, a22 - d * b12
        n2 = jnp.maximum(u0 * u0 + u1 * u1 + u2 * u2, 1e-24)
        s2 = _rsqrt(n2)
        b20, b21, b22 = u0 * s2, u1 * s2, u2 * s2

        b30 = b11 * b22 - b12 * b21
        b31 = b12 * b20 - b10 * b22
        b32 = b10 * b21 - b11 * b20

        outs = (b10, b11, b12, t0, b20, b21, b22, t1, b30, b31, b32, t2)
        for c, v in enumerate(outs):
            out_v[pl.ds(c * B_PER_W + i * L, L)] = v
        return _

    lax.fori_loop(0, GROUPS, group, None)

    pltpu.sync_copy(out_v, out_hbm.at[pl.ds(wid * 12 * B_PER_W, 12 * B_PER_W)])


@jax.jit
def _pose_kernel(table, idx2d):
    comps = _split_components(table.T)
    mesh = plsc.VectorSubcoreMesh(core_axis_name="c", subcore_axis_name="s")
    return pl.kernel(
        _pose_body,
        out_type=jax.ShapeDtypeStruct((BATCH * 12,), jnp.float32),
        mesh=mesh,
        scratch_types=[
            pltpu.VMEM((N_CHUNKS, IDX_CHUNK), jnp.int32),
            pltpu.VMEM((9 * B_PER_W,), jnp.float32),
            pltpu.VMEM((12 * B_PER_W,), jnp.float32),
            pltpu.SemaphoreType.DMA,
        ],
    )(*comps, idx2d)


def kernel(poses_embed, pose_indices):
    idx2d = pose_indices.astype(jnp.int32).reshape(BATCH // IDX_CHUNK, IDX_CHUNK)
    out = _pose_kernel(poses_embed, idx2d)
    out = out.reshape(NW, 12, B_PER_W).transpose(0, 2, 1)
    return out.reshape(BATCH, 3, 4)
